# baseline (speedup 1.0000x reference)
import functools

import jax
import jax.numpy as jnp
from jax import lax
from jax.experimental import pallas as pl
from jax.experimental.pallas import tpu as pltpu
from jax.experimental.pallas import tpu_sc as plsc

N = 10000
E = 320000
D_IN = 128
H = 64

NC = 2
NS = 16
NW = NC * NS
EPW = E // NW
NPAD = 10240
ROWS_PER_TILE = NPAD // NS

_SC_MESH = plsc.VectorSubcoreMesh(
    core_axis_name="c", subcore_axis_name="s", num_cores=NC, num_subcores=NS)

def _zero_fill(buf, nrows, ncol16):
  z16 = jnp.zeros((16,), jnp.float32)
  def zrow(i, _):
    for j in range(ncol16):
      buf[i, pl.ds(j * 16, 16)] = z16
    return 0
  lax.fori_loop(0, nrows, zrow, 0)


def _make_agg(Hp, CH):
  nch = EPW // CH
  npair = (nch + 1) // 2

  @functools.partial(
      pl.kernel,
      out_type=jax.ShapeDtypeStruct((NC, NPAD, Hp), jnp.float32),
      mesh=_SC_MESH,
      scratch_types=[
          pltpu.VMEM((2, 2, CH), jnp.int32),
          pltpu.VMEM((CH, Hp), jnp.float32),
          pltpu.VMEM((CH, Hp), jnp.float32),
          pltpu.VMEM_SHARED((NPAD, Hp), jnp.float32),
          pltpu.SemaphoreType.DMA,
          pltpu.SemaphoreType.DMA,
      ],
      compiler_params=pltpu.CompilerParams(use_tc_tiling_on_sc=False),
  )
  def agg(src_hbm, dst_hbm, y_hbm, out_hbm, idx_v, rows0, rows1, acc_sh,
          sem0, sem1):
    rows = (rows0, rows1)
    sems = (sem0, sem1)
    c = lax.axis_index("c")
    s = lax.axis_index("s")
    wid = s * NC + c

    _zero_fill(rows0, CH, Hp // 16)
    _zero_fill(rows1, ROWS_PER_TILE - CH, Hp // 16)
    pltpu.sync_copy(rows0, acc_sh.at[pl.ds(s * ROWS_PER_TILE, CH)])
    pltpu.sync_copy(
        rows1.at[pl.ds(0, ROWS_PER_TILE - CH)],
        acc_sh.at[pl.ds(s * ROWS_PER_TILE + CH, ROWS_PER_TILE - CH)])
    plsc.subcore_barrier()

    base = wid * EPW

    pltpu.sync_copy(src_hbm.at[pl.ds(base, CH)], idx_v.at[0, 0])
    pltpu.sync_copy(dst_hbm.at[pl.ds(base, CH)], idx_v.at[0, 1])
    pltpu.async_copy(y_hbm.at[idx_v.at[0, 0]], rows0, sem0)

    def pair(k, _):
      for b in range(2):
        g = 2 * k + b

        @pl.when(g + 1 < nch)
        def _prefetch():
          off = base + (g + 1) * CH
          pltpu.sync_copy(src_hbm.at[pl.ds(off, CH)], idx_v.at[1 - b, 0])
          pltpu.sync_copy(dst_hbm.at[pl.ds(off, CH)], idx_v.at[1 - b, 1])
          pltpu.async_copy(y_hbm.at[idx_v.at[1 - b, 0]], rows[1 - b],
                           sems[1 - b])

        @pl.when(g < nch)
        def _complete():
          pltpu.make_async_copy(y_hbm.at[idx_v.at[b, 0]], rows[b],
                                sems[b]).wait()
          pltpu.sync_copy(rows[b], acc_sh.at[idx_v.at[b, 1]], add=True)

      return 0

    lax.fori_loop(0, npair, pair, 0)
    plsc.subcore_barrier()

    pltpu.sync_copy(
        acc_sh.at[pl.ds(s * ROWS_PER_TILE, ROWS_PER_TILE)],
        out_hbm.at[c, pl.ds(s * ROWS_PER_TILE, ROWS_PER_TILE)])

  return agg


_agg64 = _make_agg(64, 400)
_agg16 = _make_agg(16, 400)

_DEG_CH = 2000


@functools.partial(
    pl.kernel,
    out_type=jax.ShapeDtypeStruct((NC, NPAD, 16), jnp.float32),
    mesh=_SC_MESH,
    scratch_types=[
        pltpu.VMEM((_DEG_CH,), jnp.int32),
        pltpu.VMEM((_DEG_CH, 16), jnp.float32),
        pltpu.VMEM_SHARED((NPAD, 16), jnp.float32),
    ],
    compiler_params=pltpu.CompilerParams(use_tc_tiling_on_sc=False),
)
def _deg(dst_hbm, out_hbm, dst_v, ones_v, acc_sh):
  c = lax.axis_index("c")
  s = lax.axis_index("s")
  wid = s * NC + c

  _zero_fill(ones_v, ROWS_PER_TILE, 1)
  pltpu.sync_copy(
      ones_v.at[pl.ds(0, ROWS_PER_TILE)],
      acc_sh.at[pl.ds(s * ROWS_PER_TILE, ROWS_PER_TILE)])

  onehot = jnp.where(lax.iota(jnp.int32, 16) == 0, 1.0, 0.0)

  def orow(i, _):
    ones_v[i, pl.ds(0, 16)] = onehot
    return 0

  lax.fori_loop(0, _DEG_CH, orow, 0)
  plsc.subcore_barrier()

  base = wid * EPW

  def chunk(t, _):
    off = base + t * _DEG_CH
    pltpu.sync_copy(dst_hbm.at[pl.ds(off, _DEG_CH)], dst_v)
    pltpu.sync_copy(ones_v, acc_sh.at[dst_v], add=True)
    return 0

  lax.fori_loop(0, EPW // _DEG_CH, chunk, 0)
  plsc.subcore_barrier()

  pltpu.sync_copy(
      acc_sh.at[pl.ds(s * ROWS_PER_TILE, ROWS_PER_TILE)],
      out_hbm.at[c, pl.ds(s * ROWS_PER_TILE, ROWS_PER_TILE)])


BN = 1000


def _tc1_body(x_ref, w1l_ref, w1r_ref, y1_ref, r1_ref):
  x = x_ref[...]
  y1_ref[...] = jnp.dot(x, w1l_ref[...], preferred_element_type=jnp.float32)
  r1_ref[...] = jnp.dot(x, w1r_ref[...], preferred_element_type=jnp.float32)


def _tc_mid1_body(p_ref, pdeg_ref, r_ref, b1l_ref, b1r_ref, w2l_ref, w2r_ref,
                  y2_ref, r2_ref, inv_ref):
  agg = p_ref[0] + p_ref[1]
  deg = lax.slice(pdeg_ref[0] + pdeg_ref[1], (0, 0), (BN, 1))
  inv = 1.0 / jnp.maximum(deg, 1.0)
  mean = agg * inv
  h = jnp.maximum(mean + b1l_ref[...] + r_ref[...] + b1r_ref[...], 0.0)
  y2_ref[...] = jnp.dot(h, w2l_ref[...], preferred_element_type=jnp.float32)
  r2_ref[...] = jnp.dot(h, w2r_ref[...], preferred_element_type=jnp.float32)
  inv_ref[...] = inv


def _tc_mid2_body(p_ref, inv_ref, r_ref, b2l_ref, b2r_ref, w3lp_ref, w3r_ref,
                  y3p_ref, r3_ref):
  mean = (p_ref[0] + p_ref[1]) * inv_ref[...]
  h = jnp.maximum(mean + b2l_ref[...] + r_ref[...] + b2r_ref[...], 0.0)
  y3p_ref[...] = jnp.dot(h, w3lp_ref[...], preferred_element_type=jnp.float32)
  r3_ref[...] = jnp.dot(h, w3r_ref[...], preferred_element_type=jnp.float32)


def _tc_final_body(p_ref, inv_ref, r3_ref, b3l_ref, b3r_ref, out_ref):
  s = p_ref[0] + p_ref[1]
  val = lax.slice(s, (0, 0), (BN, 1))
  out_ref[...] = val * inv_ref[...] + b3l_ref[...] + r3_ref[...] + b3r_ref[...]


def _rows(bs):
  return pl.BlockSpec(bs, lambda i: (i,) + (0,) * (len(bs) - 1))


def _whole(bs):
  return pl.BlockSpec(bs, lambda i: (0,) * len(bs))


def _prow(bs):
  return pl.BlockSpec(bs, lambda i: (0, i, 0))


def kernel(x, edge_index, W1l, b1l, W1r, b1r, W2l, b2l, W2r, b2r,
           W3l, b3l, W3r, b3r):
  src = edge_index[0]
  dst = edge_index[1]
  w3lp = jnp.pad(W3l, ((0, 0), (0, 15)))

  f32 = jnp.float32
  G = N // BN

  y1, r1 = pl.pallas_call(
      _tc1_body,
      grid=(G,),
      in_specs=[_rows((BN, D_IN)), _whole((D_IN, H)), _whole((D_IN, H))],
      out_specs=[_rows((BN, H)), _rows((BN, H))],
      out_shape=[jax.ShapeDtypeStruct((N, H), f32),
                 jax.ShapeDtypeStruct((N, H), f32)],
  )(x, W1l, W1r)

  pdeg = _deg(dst)
  p1 = _agg64(src, dst, y1)

  y2, r2, inv = pl.pallas_call(
      _tc_mid1_body,
      grid=(G,),
      in_specs=[_prow((NC, BN, H)), _prow((NC, BN, 16)), _rows((BN, H)),
                _whole((H,)), _whole((H,)), _whole((H, H)), _whole((H, H))],
      out_specs=[_rows((BN, H)), _rows((BN, H)), _rows((BN, 1))],
      out_shape=[jax.ShapeDtypeStruct((N, H), f32),
                 jax.ShapeDtypeStruct((N, H), f32),
                 jax.ShapeDtypeStruct((N, 1), f32)],
  )(p1, pdeg, r1, b1l, b1r, W2l, W2r)

  p2 = _agg64(src, dst, y2)

  y3p, r3 = pl.pallas_call(
      _tc_mid2_body,
      grid=(G,),
      in_specs=[_prow((NC, BN, H)), _rows((BN, 1)), _rows((BN, H)),
                _whole((H,)), _whole((H,)), _whole((H, 16)), _whole((H, 1))],
      out_specs=[_rows((BN, 16)), _rows((BN, 1))],
      out_shape=[jax.ShapeDtypeStruct((N, 16), f32),
                 jax.ShapeDtypeStruct((N, 1), f32)],
  )(p2, inv, r2, b2l, b2r, w3lp, W3r)

  p3 = _agg16(src, dst, y3p)

  out = pl.pallas_call(
      _tc_final_body,
      grid=(G,),
      in_specs=[_prow((NC, BN, 16)), _rows((BN, 1)), _rows((BN, 1)),
                _whole((1,)), _whole((1,))],
      out_specs=_rows((BN, 1)),
      out_shape=jax.ShapeDtypeStruct((N, 1), f32),
  )(p3, inv, r3, b3l, b3r)

  return out.reshape(N)


# speedup vs baseline: 1.2565x; 1.2565x over previous
import functools

import jax
import jax.numpy as jnp
from jax import lax
from jax.experimental import pallas as pl
from jax.experimental.pallas import tpu as pltpu
from jax.experimental.pallas import tpu_sc as plsc

N = 10000
E = 320000
D_IN = 128
H = 64

NC = 2
NS = 16
NW = NC * NS
EPW = E // NW
NPAD = 10240
ROWS_PER_TILE = NPAD // NS

_SC_MESH = plsc.VectorSubcoreMesh(
    core_axis_name="c", subcore_axis_name="s", num_cores=NC, num_subcores=NS)


def _fill(buf, nrows, ncol16, vec):
  assert nrows % 4 == 0
  def frow(i, _):
    for u in range(4):
      for j in range(ncol16):
        buf[i * 4 + u, pl.ds(j * 16, 16)] = vec
    return 0
  lax.fori_loop(0, nrows // 4, frow, 0)


def _make_agg(Hp, CH):
  nch = EPW // CH
  NB = 3
  ngrp = (nch + NB - 1) // NB

  @functools.partial(
      pl.kernel,
      out_type=jax.ShapeDtypeStruct((NC, NPAD, Hp), jnp.float32),
      mesh=_SC_MESH,
      scratch_types=[
          pltpu.VMEM((NB, 2, CH), jnp.int32),
          pltpu.VMEM((CH, Hp), jnp.float32),
          pltpu.VMEM((CH, Hp), jnp.float32),
          pltpu.VMEM((CH, Hp), jnp.float32),
          pltpu.VMEM_SHARED((NPAD, Hp), jnp.float32),
          pltpu.SemaphoreType.DMA,
          pltpu.SemaphoreType.DMA,
          pltpu.SemaphoreType.DMA,
          pltpu.SemaphoreType.DMA,
          pltpu.SemaphoreType.DMA,
          pltpu.SemaphoreType.DMA,
      ],
      compiler_params=pltpu.CompilerParams(use_tc_tiling_on_sc=False),
  )
  def agg(ei_hbm, y_hbm, out_hbm, idx_v, rows0, rows1, rows2, acc_sh,
          gsem0, gsem1, gsem2, ssem0, ssem1, ssem2):
    rows = (rows0, rows1, rows2)
    gsems = (gsem0, gsem1, gsem2)
    ssems = (ssem0, ssem1, ssem2)
    c = lax.axis_index("c")
    s = lax.axis_index("s")
    wid = s * NC + c
    z16 = jnp.zeros((16,), jnp.float32)

    _fill(rows0, CH, Hp // 16, z16)
    _fill(rows1, ROWS_PER_TILE - CH, Hp // 16, z16)
    pltpu.sync_copy(rows0, acc_sh.at[pl.ds(s * ROWS_PER_TILE, CH)])
    pltpu.sync_copy(
        rows1.at[pl.ds(0, ROWS_PER_TILE - CH)],
        acc_sh.at[pl.ds(s * ROWS_PER_TILE + CH, ROWS_PER_TILE - CH)])
    plsc.subcore_barrier()

    base = wid * EPW

    def _stage(q, bq):
      off = base + q * CH
      pltpu.sync_copy(ei_hbm.at[:, pl.ds(off, CH)], idx_v.at[bq])
      pltpu.async_copy(y_hbm.at[idx_v.at[bq, 0]], rows[bq], gsems[bq])

    def _scatter_wait(b):
      pltpu.make_async_copy(rows[b], acc_sh.at[idx_v.at[b, 1]],
                            ssems[b]).wait()

    for q in range(NB - 1):
      _stage(q, q)

    def grp(k, _):
      for b in range(NB):
        g = NB * k + b

        @pl.when(g < nch)
        def _step():
          pltpu.make_async_copy(y_hbm.at[idx_v.at[b, 0]], rows[b],
                                gsems[b]).wait()
          pltpu.async_copy(rows[b], acc_sh.at[idx_v.at[b, 1]], ssems[b],
                           add=True)

          q = g + NB - 1
          bq = (b + NB - 1) % NB

          @pl.when(q < nch)
          def _prefetch():
            @pl.when(g >= 1)
            def _drain():
              _scatter_wait(bq)
            _stage(q, bq)

      return 0

    lax.fori_loop(0, ngrp, grp, 0)
    for b in range(NB):
      _scatter_wait(b)
    plsc.subcore_barrier()

    pltpu.sync_copy(
        acc_sh.at[pl.ds(s * ROWS_PER_TILE, ROWS_PER_TILE)],
        out_hbm.at[c, pl.ds(s * ROWS_PER_TILE, ROWS_PER_TILE)])

  return agg


_agg64 = _make_agg(64, 400)
_agg16 = _make_agg(16, 400)

_DEG_CH = 2000


@functools.partial(
    pl.kernel,
    out_type=jax.ShapeDtypeStruct((NC, NPAD, 16), jnp.float32),
    mesh=_SC_MESH,
    scratch_types=[
        pltpu.VMEM((_DEG_CH,), jnp.int32),
        pltpu.VMEM((_DEG_CH, 16), jnp.float32),
        pltpu.VMEM_SHARED((NPAD, 16), jnp.float32),
    ],
    compiler_params=pltpu.CompilerParams(use_tc_tiling_on_sc=False),
)
def _deg(ei_hbm, out_hbm, dst_v, ones_v, acc_sh):
  c = lax.axis_index("c")
  s = lax.axis_index("s")
  wid = s * NC + c

  z16 = jnp.zeros((16,), jnp.float32)
  _fill(ones_v, ROWS_PER_TILE, 1, z16)
  pltpu.sync_copy(
      ones_v.at[pl.ds(0, ROWS_PER_TILE)],
      acc_sh.at[pl.ds(s * ROWS_PER_TILE, ROWS_PER_TILE)])

  onehot = jnp.where(lax.iota(jnp.int32, 16) == 0, 1.0, 0.0)
  _fill(ones_v, _DEG_CH, 1, onehot)
  plsc.subcore_barrier()

  base = wid * EPW

  def chunk(t, _):
    off = base + t * _DEG_CH
    pltpu.sync_copy(ei_hbm.at[1, pl.ds(off, _DEG_CH)], dst_v)
    pltpu.sync_copy(ones_v, acc_sh.at[dst_v], add=True)
    return 0

  lax.fori_loop(0, EPW // _DEG_CH, chunk, 0)
  plsc.subcore_barrier()

  pltpu.sync_copy(
      acc_sh.at[pl.ds(s * ROWS_PER_TILE, ROWS_PER_TILE)],
      out_hbm.at[c, pl.ds(s * ROWS_PER_TILE, ROWS_PER_TILE)])


def _unpack(p_ref, Hp):
  a = p_ref[0] + p_ref[1]
  return lax.slice(a, (0, 0), (N, Hp))


def _tc1_body(x_ref, w1l_ref, w1r_ref, y1_ref, r1_ref):
  x = x_ref[...]
  y1_ref[...] = jnp.dot(x, w1l_ref[...], preferred_element_type=jnp.float32)
  r1_ref[...] = jnp.dot(x, w1r_ref[...], preferred_element_type=jnp.float32)


def _tc_mid1_body(p_ref, pdeg_ref, r_ref, b1l_ref, b1r_ref, w2l_ref, w2r_ref,
                  y2_ref, r2_ref, inv_ref):
  agg = _unpack(p_ref, H)
  deg = lax.slice(_unpack(pdeg_ref, 16), (0, 0), (N, 1))
  inv = 1.0 / jnp.maximum(deg, 1.0)
  mean = agg * inv
  h = jnp.maximum(mean + b1l_ref[...] + r_ref[...] + b1r_ref[...], 0.0)
  y2_ref[...] = jnp.dot(h, w2l_ref[...], preferred_element_type=jnp.float32)
  r2_ref[...] = jnp.dot(h, w2r_ref[...], preferred_element_type=jnp.float32)
  inv_ref[...] = inv


def _tc_mid2_body(p_ref, inv_ref, r_ref, b2l_ref, b2r_ref, w3lp_ref, w3r_ref,
                  y3p_ref, r3_ref):
  mean = _unpack(p_ref, H) * inv_ref[...]
  h = jnp.maximum(mean + b2l_ref[...] + r_ref[...] + b2r_ref[...], 0.0)
  y3p_ref[...] = jnp.dot(h, w3lp_ref[...], preferred_element_type=jnp.float32)
  r3_ref[...] = jnp.dot(h, w3r_ref[...], preferred_element_type=jnp.float32)


def _tc_final_body(p_ref, inv_ref, r3_ref, b3l_ref, b3r_ref, out_ref):
  val = lax.slice(_unpack(p_ref, 16), (0, 0), (N, 1))
  out_ref[...] = val * inv_ref[...] + b3l_ref[...] + r3_ref[...] + b3r_ref[...]


def kernel(x, edge_index, W1l, b1l, W1r, b1r, W2l, b2l, W2r, b2r,
           W3l, b3l, W3r, b3r):
  w3lp = jnp.pad(W3l, ((0, 0), (0, 15)))

  f32 = jnp.float32
  pdeg = _deg(edge_index)

  y1, r1 = pl.pallas_call(
      _tc1_body,
      out_shape=[jax.ShapeDtypeStruct((N, H), f32),
                 jax.ShapeDtypeStruct((N, H), f32)],
  )(x, W1l, W1r)

  p1 = _agg64(edge_index, y1)

  y2, r2, inv = pl.pallas_call(
      _tc_mid1_body,
      out_shape=[jax.ShapeDtypeStruct((N, H), f32),
                 jax.ShapeDtypeStruct((N, H), f32),
                 jax.ShapeDtypeStruct((N, 1), f32)],
  )(p1, pdeg, r1, b1l, b1r, W2l, W2r)

  p2 = _agg64(edge_index, y2)

  y3p, r3 = pl.pallas_call(
      _tc_mid2_body,
      out_shape=[jax.ShapeDtypeStruct((N, 16), f32),
                 jax.ShapeDtypeStruct((N, 1), f32)],
  )(p2, inv, r2, b2l, b2r, w3lp, W3r)

  p3 = _agg16(edge_index, y3p)

  out = pl.pallas_call(
      _tc_final_body,
      out_shape=jax.ShapeDtypeStruct((N, 1), f32),
  )(p3, inv, r3, b3l, b3r)

  return out.reshape(N)


# speedup vs baseline: 1.2719x; 1.0122x over previous
import functools

import jax
import jax.numpy as jnp
from jax import lax
from jax.experimental import pallas as pl
from jax.experimental.pallas import tpu as pltpu
from jax.experimental.pallas import tpu_sc as plsc

N = 10000
E = 320000
D_IN = 128
H = 64

NC = 2
NS = 16
NW = NC * NS
EPW = E // NW
NPAD = 10240
ROWS_PER_TILE = NPAD // NS

_SC_MESH = plsc.VectorSubcoreMesh(
    core_axis_name="c", subcore_axis_name="s", num_cores=NC, num_subcores=NS)


def _fill(buf, nrows, ncol16, vec):
  assert nrows % 4 == 0
  def frow(i, _):
    for u in range(4):
      for j in range(ncol16):
        buf[i * 4 + u, pl.ds(j * 16, 16)] = vec
    return 0
  lax.fori_loop(0, nrows // 4, frow, 0)


def _make_agg(Hp, CH):
  nch = EPW // CH
  NB = 3
  ngrp = (nch + NB - 1) // NB

  @functools.partial(
      pl.kernel,
      out_type=jax.ShapeDtypeStruct((NC, NPAD, Hp), jnp.float32),
      mesh=_SC_MESH,
      scratch_types=[
          pltpu.VMEM((NB, 2, CH), jnp.int32),
          pltpu.VMEM((CH, Hp), jnp.float32),
          pltpu.VMEM((CH, Hp), jnp.float32),
          pltpu.VMEM((CH, Hp), jnp.float32),
          pltpu.VMEM_SHARED((NPAD, Hp), jnp.float32),
          pltpu.SemaphoreType.DMA,
          pltpu.SemaphoreType.DMA,
          pltpu.SemaphoreType.DMA,
          pltpu.SemaphoreType.DMA,
          pltpu.SemaphoreType.DMA,
          pltpu.SemaphoreType.DMA,
      ],
      compiler_params=pltpu.CompilerParams(use_tc_tiling_on_sc=False),
  )
  def agg(ei_hbm, y_hbm, out_hbm, idx_v, rows0, rows1, rows2, acc_sh,
          gsem0, gsem1, gsem2, ssem0, ssem1, ssem2):
    rows = (rows0, rows1, rows2)
    gsems = (gsem0, gsem1, gsem2)
    ssems = (ssem0, ssem1, ssem2)
    c = lax.axis_index("c")
    s = lax.axis_index("s")
    wid = s * NC + c
    z16 = jnp.zeros((16,), jnp.float32)

    _fill(rows0, CH, Hp // 16, z16)
    _fill(rows1, ROWS_PER_TILE - CH, Hp // 16, z16)
    pltpu.sync_copy(rows0, acc_sh.at[pl.ds(s * ROWS_PER_TILE, CH)])
    pltpu.sync_copy(
        rows1.at[pl.ds(0, ROWS_PER_TILE - CH)],
        acc_sh.at[pl.ds(s * ROWS_PER_TILE + CH, ROWS_PER_TILE - CH)])
    plsc.subcore_barrier()

    base = wid * EPW

    def _stage(q, bq):
      off = base + q * CH
      pltpu.sync_copy(ei_hbm.at[:, pl.ds(off, CH)], idx_v.at[bq])
      pltpu.async_copy(y_hbm.at[idx_v.at[bq, 0]], rows[bq], gsems[bq])

    def _scatter_wait(b):
      pltpu.make_async_copy(rows[b], acc_sh.at[idx_v.at[b, 1]],
                            ssems[b]).wait()

    for q in range(NB - 1):
      _stage(q, q)

    def grp(k, _):
      for b in range(NB):
        g = NB * k + b

        @pl.when(g < nch)
        def _step():
          pltpu.make_async_copy(y_hbm.at[idx_v.at[b, 0]], rows[b],
                                gsems[b]).wait()
          pltpu.async_copy(rows[b], acc_sh.at[idx_v.at[b, 1]], ssems[b],
                           add=True)

          q = g + NB - 1
          bq = (b + NB - 1) % NB

          @pl.when(q < nch)
          def _prefetch():
            @pl.when(g >= 1)
            def _drain():
              _scatter_wait(bq)
            _stage(q, bq)

      return 0

    lax.fori_loop(0, ngrp, grp, 0)
    for b in range(NB):
      _scatter_wait(b)
    plsc.subcore_barrier()

    pltpu.sync_copy(
        acc_sh.at[pl.ds(s * ROWS_PER_TILE, ROWS_PER_TILE)],
        out_hbm.at[c, pl.ds(s * ROWS_PER_TILE, ROWS_PER_TILE)])

  return agg


_agg64_l1 = _make_agg(64, 400)
_agg64_l2 = _make_agg(64, 400)
_agg16 = _make_agg(16, 400)

_DEG_CH = 2000


@functools.partial(
    pl.kernel,
    out_type=jax.ShapeDtypeStruct((NC, NPAD, 16), jnp.float32),
    mesh=_SC_MESH,
    scratch_types=[
        pltpu.VMEM((_DEG_CH,), jnp.int32),
        pltpu.VMEM((_DEG_CH, 16), jnp.float32),
        pltpu.VMEM_SHARED((NPAD, 16), jnp.float32),
    ],
    compiler_params=pltpu.CompilerParams(use_tc_tiling_on_sc=False),
)
def _deg(ei_hbm, out_hbm, dst_v, ones_v, acc_sh):
  c = lax.axis_index("c")
  s = lax.axis_index("s")
  wid = s * NC + c

  z16 = jnp.zeros((16,), jnp.float32)
  _fill(ones_v, ROWS_PER_TILE, 1, z16)
  pltpu.sync_copy(
      ones_v.at[pl.ds(0, ROWS_PER_TILE)],
      acc_sh.at[pl.ds(s * ROWS_PER_TILE, ROWS_PER_TILE)])

  onehot = jnp.where(lax.iota(jnp.int32, 16) == 0, 1.0, 0.0)
  _fill(ones_v, _DEG_CH, 1, onehot)
  plsc.subcore_barrier()

  base = wid * EPW

  def chunk(t, _):
    off = base + t * _DEG_CH
    pltpu.sync_copy(ei_hbm.at[1, pl.ds(off, _DEG_CH)], dst_v)
    pltpu.sync_copy(ones_v, acc_sh.at[dst_v], add=True)
    return 0

  lax.fori_loop(0, EPW // _DEG_CH, chunk, 0)
  plsc.subcore_barrier()

  pltpu.sync_copy(
      acc_sh.at[pl.ds(s * ROWS_PER_TILE, ROWS_PER_TILE)],
      out_hbm.at[c, pl.ds(s * ROWS_PER_TILE, ROWS_PER_TILE)])


def _unpack(p_ref, Hp):
  a = p_ref[0] + p_ref[1]
  return lax.slice(a, (0, 0), (N, Hp))


def _tc1_body(x_ref, w1l_ref, w1r_ref, y1_ref, r1_ref):
  x = x_ref[...]
  y1_ref[...] = jnp.dot(x, w1l_ref[...], preferred_element_type=jnp.float32)
  r1_ref[...] = jnp.dot(x, w1r_ref[...], preferred_element_type=jnp.float32)


def _tc_mid1_body(p_ref, pdeg_ref, r_ref, b1l_ref, b1r_ref, w2l_ref, w2r_ref,
                  y2_ref, r2_ref, inv_ref):
  agg = _unpack(p_ref, H)
  deg = lax.slice(_unpack(pdeg_ref, 16), (0, 0), (N, 1))
  inv = 1.0 / jnp.maximum(deg, 1.0)
  mean = agg * inv
  h = jnp.maximum(mean + b1l_ref[...] + r_ref[...] + b1r_ref[...], 0.0)
  y2_ref[...] = jnp.dot(h, w2l_ref[...], preferred_element_type=jnp.float32)
  r2_ref[...] = jnp.dot(h, w2r_ref[...], preferred_element_type=jnp.float32)
  inv_ref[...] = inv


def _tc_mid2_body(p_ref, inv_ref, r_ref, b2l_ref, b2r_ref, w3lp_ref, w3r_ref,
                  y3p_ref, r3_ref):
  mean = _unpack(p_ref, H) * inv_ref[...]
  h = jnp.maximum(mean + b2l_ref[...] + r_ref[...] + b2r_ref[...], 0.0)
  y3p_ref[...] = jnp.dot(h, w3lp_ref[...], preferred_element_type=jnp.float32)
  r3_ref[...] = jnp.dot(h, w3r_ref[...], preferred_element_type=jnp.float32)


def _tc_final_body(p_ref, inv_ref, r3_ref, b3l_ref, b3r_ref, out_ref):
  val = lax.slice(_unpack(p_ref, 16), (0, 0), (N, 1))
  res = val * inv_ref[...] + b3l_ref[...] + r3_ref[...] + b3r_ref[...]
  out_ref[...] = jnp.reshape(res, (N,))


def kernel(x, edge_index, W1l, b1l, W1r, b1r, W2l, b2l, W2r, b2r,
           W3l, b3l, W3r, b3r):
  w3lp = jnp.pad(W3l, ((0, 0), (0, 15)))

  f32 = jnp.float32
  pdeg = _deg(edge_index)

  y1, r1 = pl.pallas_call(
      _tc1_body,
      out_shape=[jax.ShapeDtypeStruct((N, H), f32),
                 jax.ShapeDtypeStruct((N, H), f32)],
  )(x, W1l, W1r)

  p1 = _agg64_l1(edge_index, y1)

  y2, r2, inv = pl.pallas_call(
      _tc_mid1_body,
      out_shape=[jax.ShapeDtypeStruct((N, H), f32),
                 jax.ShapeDtypeStruct((N, H), f32),
                 jax.ShapeDtypeStruct((N, 1), f32)],
  )(p1, pdeg, r1, b1l, b1r, W2l, W2r)

  p2 = _agg64_l2(edge_index, y2)

  y3p, r3 = pl.pallas_call(
      _tc_mid2_body,
      out_shape=[jax.ShapeDtypeStruct((N, 16), f32),
                 jax.ShapeDtypeStruct((N, 1), f32)],
  )(p2, inv, r2, b2l, b2r, w3lp, W3r)

  p3 = _agg16(edge_index, y3p)

  out = pl.pallas_call(
      _tc_final_body,
      out_shape=jax.ShapeDtypeStruct((N,), f32),
  )(p3, inv, r3, b3l, b3r)

  return out


# speedup vs baseline: 1.2795x; 1.0060x over previous
import functools

import jax
import jax.numpy as jnp
from jax import lax
from jax.experimental import pallas as pl
from jax.experimental.pallas import tpu as pltpu
from jax.experimental.pallas import tpu_sc as plsc

N = 10000
E = 320000
D_IN = 128
H = 64

NC = 2
NS = 16
NW = NC * NS
EPW = E // NW
NPAD = 10240
ROWS_PER_TILE = NPAD // NS

_SC_MESH = plsc.VectorSubcoreMesh(
    core_axis_name="c", subcore_axis_name="s", num_cores=NC, num_subcores=NS)


def _fill(buf, nrows, ncol16, vec):
  assert nrows % 4 == 0
  def frow(i, _):
    for u in range(4):
      for j in range(ncol16):
        buf[i * 4 + u, pl.ds(j * 16, 16)] = vec
    return 0
  lax.fori_loop(0, nrows // 4, frow, 0)


def _make_agg(Hp, CH):
  nch = EPW // CH
  NB = 3
  ngrp = (nch + NB - 1) // NB

  @functools.partial(
      pl.kernel,
      out_type=jax.ShapeDtypeStruct((NC, NPAD, Hp), jnp.float32),
      mesh=_SC_MESH,
      scratch_types=[
          pltpu.VMEM((NB, 2, CH), jnp.int32),
          pltpu.VMEM((CH, Hp), jnp.float32),
          pltpu.VMEM((CH, Hp), jnp.float32),
          pltpu.VMEM((CH, Hp), jnp.float32),
          pltpu.VMEM_SHARED((NPAD, Hp), jnp.float32),
          pltpu.SemaphoreType.DMA,
          pltpu.SemaphoreType.DMA,
          pltpu.SemaphoreType.DMA,
          pltpu.SemaphoreType.DMA,
          pltpu.SemaphoreType.DMA,
          pltpu.SemaphoreType.DMA,
      ],
      compiler_params=pltpu.CompilerParams(use_tc_tiling_on_sc=False),
  )
  def agg(ei_hbm, y_hbm, out_hbm, idx_v, rows0, rows1, rows2, acc_sh,
          gsem0, gsem1, gsem2, ssem0, ssem1, ssem2):
    rows = (rows0, rows1, rows2)
    gsems = (gsem0, gsem1, gsem2)
    ssems = (ssem0, ssem1, ssem2)
    c = lax.axis_index("c")
    s = lax.axis_index("s")
    wid = s * NC + c
    z16 = jnp.zeros((16,), jnp.float32)

    _fill(rows0, CH, Hp // 16, z16)
    _fill(rows1, ROWS_PER_TILE - CH, Hp // 16, z16)
    pltpu.sync_copy(rows0, acc_sh.at[pl.ds(s * ROWS_PER_TILE, CH)])
    pltpu.sync_copy(
        rows1.at[pl.ds(0, ROWS_PER_TILE - CH)],
        acc_sh.at[pl.ds(s * ROWS_PER_TILE + CH, ROWS_PER_TILE - CH)])
    plsc.subcore_barrier()

    base = wid * EPW

    def _stage(q, bq):
      off = base + q * CH
      pltpu.sync_copy(ei_hbm.at[:, pl.ds(off, CH)], idx_v.at[bq])
      pltpu.async_copy(y_hbm.at[idx_v.at[bq, 0]], rows[bq], gsems[bq])

    def _scatter_wait(b):
      pltpu.make_async_copy(rows[b], acc_sh.at[idx_v.at[b, 1]],
                            ssems[b]).wait()

    for q in range(NB - 1):
      _stage(q, q)

    def grp(k, _):
      for b in range(NB):
        g = NB * k + b

        @pl.when(g < nch)
        def _step():
          pltpu.make_async_copy(y_hbm.at[idx_v.at[b, 0]], rows[b],
                                gsems[b]).wait()
          pltpu.async_copy(rows[b], acc_sh.at[idx_v.at[b, 1]], ssems[b],
                           add=True)

          q = g + NB - 1
          bq = (b + NB - 1) % NB

          @pl.when(q < nch)
          def _prefetch():
            @pl.when(g >= 1)
            def _drain():
              _scatter_wait(bq)
            _stage(q, bq)

      return 0

    lax.fori_loop(0, ngrp, grp, 0)
    for b in range(NB):
      _scatter_wait(b)
    plsc.subcore_barrier()

    pltpu.sync_copy(
        acc_sh.at[pl.ds(s * ROWS_PER_TILE, ROWS_PER_TILE)],
        out_hbm.at[c, pl.ds(s * ROWS_PER_TILE, ROWS_PER_TILE)])

  return agg


_agg64_l1 = _make_agg(64, 400)
_agg64_l2 = _make_agg(64, 400)
_agg16 = _make_agg(16, 400)

_DEG_CH = 2000


@functools.partial(
    pl.kernel,
    out_type=jax.ShapeDtypeStruct((NC, NPAD, 16), jnp.float32),
    mesh=_SC_MESH,
    scratch_types=[
        pltpu.VMEM((_DEG_CH,), jnp.int32),
        pltpu.VMEM((_DEG_CH, 16), jnp.float32),
        pltpu.VMEM_SHARED((NPAD, 16), jnp.float32),
    ],
    compiler_params=pltpu.CompilerParams(use_tc_tiling_on_sc=False),
)
def _deg(ei_hbm, out_hbm, dst_v, ones_v, acc_sh):
  c = lax.axis_index("c")
  s = lax.axis_index("s")
  wid = s * NC + c

  z16 = jnp.zeros((16,), jnp.float32)
  _fill(ones_v, ROWS_PER_TILE, 1, z16)
  pltpu.sync_copy(
      ones_v.at[pl.ds(0, ROWS_PER_TILE)],
      acc_sh.at[pl.ds(s * ROWS_PER_TILE, ROWS_PER_TILE)])

  onehot = jnp.where(lax.iota(jnp.int32, 16) == 0, 1.0, 0.0)
  _fill(ones_v, _DEG_CH, 1, onehot)
  plsc.subcore_barrier()

  base = wid * EPW

  def chunk(t, _):
    off = base + t * _DEG_CH
    pltpu.sync_copy(ei_hbm.at[1, pl.ds(off, _DEG_CH)], dst_v)
    pltpu.sync_copy(ones_v, acc_sh.at[dst_v], add=True)
    return 0

  lax.fori_loop(0, EPW // _DEG_CH, chunk, 0)
  plsc.subcore_barrier()

  pltpu.sync_copy(
      acc_sh.at[pl.ds(s * ROWS_PER_TILE, ROWS_PER_TILE)],
      out_hbm.at[c, pl.ds(s * ROWS_PER_TILE, ROWS_PER_TILE)])


def _unpack(p_ref, Hp):
  a = p_ref[0] + p_ref[1]
  return lax.slice(a, (0, 0), (N, Hp))


def _tc_mm_body(x_ref, w_ref, y_ref):
  y_ref[...] = jnp.dot(x_ref[...], w_ref[...],
                       preferred_element_type=jnp.float32)


def _mm(x, w, dout):
  return pl.pallas_call(
      _tc_mm_body,
      out_shape=jax.ShapeDtypeStruct((N, dout), jnp.float32),
  )(x, w)


def _tc_mid1_body(p_ref, pdeg_ref, r_ref, b1l_ref, b1r_ref, w2l_ref,
                  y2_ref, h_ref, inv_ref):
  agg = _unpack(p_ref, H)
  deg = lax.slice(_unpack(pdeg_ref, 16), (0, 0), (N, 1))
  inv = 1.0 / jnp.maximum(deg, 1.0)
  mean = agg * inv
  h = jnp.maximum(mean + b1l_ref[...] + r_ref[...] + b1r_ref[...], 0.0)
  y2_ref[...] = jnp.dot(h, w2l_ref[...], preferred_element_type=jnp.float32)
  h_ref[...] = h
  inv_ref[...] = inv


def _tc_mid2_body(p_ref, inv_ref, r_ref, b2l_ref, b2r_ref, w3lp_ref, w3r_ref,
                  y3p_ref, r3_ref):
  mean = _unpack(p_ref, H) * inv_ref[...]
  h = jnp.maximum(mean + b2l_ref[...] + r_ref[...] + b2r_ref[...], 0.0)
  y3p_ref[...] = jnp.dot(h, w3lp_ref[...], preferred_element_type=jnp.float32)
  r3_ref[...] = jnp.dot(h, w3r_ref[...], preferred_element_type=jnp.float32)


def _tc_final_body(p_ref, inv_ref, r3_ref, b3l_ref, b3r_ref, out_ref):
  val = lax.slice(_unpack(p_ref, 16), (0, 0), (N, 1))
  res = val * inv_ref[...] + b3l_ref[...] + r3_ref[...] + b3r_ref[...]
  out_ref[...] = jnp.reshape(res, (N,))


def kernel(x, edge_index, W1l, b1l, W1r, b1r, W2l, b2l, W2r, b2r,
           W3l, b3l, W3r, b3r):
  w3lp = jnp.pad(W3l, ((0, 0), (0, 15)))

  f32 = jnp.float32
  y1 = _mm(x, W1l, H)
  pdeg = _deg(edge_index)

  p1 = _agg64_l1(edge_index, y1)
  r1 = _mm(x, W1r, H)

  y2, h1, inv = pl.pallas_call(
      _tc_mid1_body,
      out_shape=[jax.ShapeDtypeStruct((N, H), f32),
                 jax.ShapeDtypeStruct((N, H), f32),
                 jax.ShapeDtypeStruct((N, 1), f32)],
  )(p1, pdeg, r1, b1l, b1r, W2l)

  p2 = _agg64_l2(edge_index, y2)
  r2 = _mm(h1, W2r, H)

  y3p, r3 = pl.pallas_call(
      _tc_mid2_body,
      out_shape=[jax.ShapeDtypeStruct((N, 16), f32),
                 jax.ShapeDtypeStruct((N, 1), f32)],
  )(p2, inv, r2, b2l, b2r, w3lp, W3r)

  p3 = _agg16(edge_index, y3p)

  out = pl.pallas_call(
      _tc_final_body,
      out_shape=jax.ShapeDtypeStruct((N,), f32),
  )(p3, inv, r3, b3l, b3r)

  return out


# speedup vs baseline: 1.3046x; 1.0196x over previous
import functools

import jax
import jax.numpy as jnp
from jax import lax
from jax.experimental import pallas as pl
from jax.experimental.pallas import tpu as pltpu
from jax.experimental.pallas import tpu_sc as plsc

N = 10000
E = 320000
D_IN = 128
H = 64

NC = 2
NS = 16
NW = NC * NS
EPW = E // NW
NPAD = 10240
ROWS_PER_TILE = NPAD // NS

_SC_MESH = plsc.VectorSubcoreMesh(
    core_axis_name="c", subcore_axis_name="s", num_cores=NC, num_subcores=NS)


def _fill(buf, nrows, ncol16, vec):
  assert nrows % 4 == 0
  def frow(i, _):
    for u in range(4):
      for j in range(ncol16):
        buf[i * 4 + u, pl.ds(j * 16, 16)] = vec
    return 0
  lax.fori_loop(0, nrows // 4, frow, 0)


def _make_agg(Hp, CH):
  nch = EPW // CH
  NB = 3
  ngrp = (nch + NB - 1) // NB

  @functools.partial(
      pl.kernel,
      out_type=jax.ShapeDtypeStruct((NC, NPAD, Hp), jnp.float32),
      mesh=_SC_MESH,
      scratch_types=[
          pltpu.VMEM((NB, 2, CH), jnp.int32),
          pltpu.VMEM((CH, Hp), jnp.float32),
          pltpu.VMEM((CH, Hp), jnp.float32),
          pltpu.VMEM((CH, Hp), jnp.float32),
          pltpu.VMEM_SHARED((NPAD, Hp), jnp.float32),
          pltpu.SemaphoreType.DMA,
          pltpu.SemaphoreType.DMA,
          pltpu.SemaphoreType.DMA,
          pltpu.SemaphoreType.DMA,
          pltpu.SemaphoreType.DMA,
          pltpu.SemaphoreType.DMA,
      ],
      compiler_params=pltpu.CompilerParams(use_tc_tiling_on_sc=False),
  )
  def agg(ei_hbm, y_hbm, out_hbm, idx_v, rows0, rows1, rows2, acc_sh,
          gsem0, gsem1, gsem2, ssem0, ssem1, ssem2):
    rows = (rows0, rows1, rows2)
    gsems = (gsem0, gsem1, gsem2)
    ssems = (ssem0, ssem1, ssem2)
    c = lax.axis_index("c")
    s = lax.axis_index("s")
    wid = s * NC + c
    z16 = jnp.zeros((16,), jnp.float32)

    _fill(rows0, CH, Hp // 16, z16)
    _fill(rows1, ROWS_PER_TILE - CH, Hp // 16, z16)
    pltpu.sync_copy(rows0, acc_sh.at[pl.ds(s * ROWS_PER_TILE, CH)])
    pltpu.sync_copy(
        rows1.at[pl.ds(0, ROWS_PER_TILE - CH)],
        acc_sh.at[pl.ds(s * ROWS_PER_TILE + CH, ROWS_PER_TILE - CH)])
    plsc.subcore_barrier()

    base = wid * EPW

    def _stage(q, bq):
      off = base + q * CH
      pltpu.sync_copy(ei_hbm.at[:, pl.ds(off, CH)], idx_v.at[bq])
      pltpu.async_copy(y_hbm.at[idx_v.at[bq, 0]], rows[bq], gsems[bq])

    def _scatter_wait(b):
      pltpu.make_async_copy(rows[b], acc_sh.at[idx_v.at[b, 1]],
                            ssems[b]).wait()

    for q in range(NB - 1):
      _stage(q, q)

    def grp(k, _):
      for b in range(NB):
        g = NB * k + b

        @pl.when(g < nch)
        def _step():
          pltpu.make_async_copy(y_hbm.at[idx_v.at[b, 0]], rows[b],
                                gsems[b]).wait()
          pltpu.async_copy(rows[b], acc_sh.at[idx_v.at[b, 1]], ssems[b],
                           add=True)

          q = g + NB - 1
          bq = (b + NB - 1) % NB

          @pl.when(q < nch)
          def _prefetch():
            @pl.when(g >= 1)
            def _drain():
              _scatter_wait(bq)
            _stage(q, bq)

      return 0

    lax.fori_loop(0, ngrp, grp, 0)
    for b in range(NB):
      _scatter_wait(b)
    plsc.subcore_barrier()

    pltpu.sync_copy(
        acc_sh.at[pl.ds(s * ROWS_PER_TILE, ROWS_PER_TILE)],
        out_hbm.at[c, pl.ds(s * ROWS_PER_TILE, ROWS_PER_TILE)])

  return agg


_agg64_l1 = _make_agg(64, 400)
_agg64_l2 = _make_agg(64, 400)
_agg16 = _make_agg(16, 400)

_DEG_CH = 2000


@functools.partial(
    pl.kernel,
    out_type=[jax.ShapeDtypeStruct((NC, NPAD, 16), jnp.float32),
              jax.ShapeDtypeStruct((2, E), jnp.int32)],
    mesh=_SC_MESH,
    scratch_types=[
        pltpu.VMEM((2, _DEG_CH), jnp.int32),
        pltpu.VMEM((_DEG_CH, 16), jnp.float32),
        pltpu.VMEM_SHARED((NPAD, 16), jnp.float32),
    ],
    compiler_params=pltpu.CompilerParams(use_tc_tiling_on_sc=False),
)
def _deg(ei_hbm, out_hbm, eiu_hbm, idx2_v, ones_v, acc_sh):
  c = lax.axis_index("c")
  s = lax.axis_index("s")
  wid = s * NC + c

  z16 = jnp.zeros((16,), jnp.float32)
  _fill(ones_v, ROWS_PER_TILE, 1, z16)
  pltpu.sync_copy(
      ones_v.at[pl.ds(0, ROWS_PER_TILE)],
      acc_sh.at[pl.ds(s * ROWS_PER_TILE, ROWS_PER_TILE)])

  onehot = jnp.where(lax.iota(jnp.int32, 16) == 0, 1.0, 0.0)
  _fill(ones_v, _DEG_CH, 1, onehot)
  plsc.subcore_barrier()

  base = wid * EPW

  def chunk(t, _):
    off = base + t * _DEG_CH
    pltpu.sync_copy(ei_hbm.at[:, pl.ds(off, _DEG_CH)], idx2_v)
    pltpu.sync_copy(idx2_v, eiu_hbm.at[:, pl.ds(off, _DEG_CH)])
    pltpu.sync_copy(ones_v, acc_sh.at[idx2_v.at[1]], add=True)
    return 0

  lax.fori_loop(0, EPW // _DEG_CH, chunk, 0)
  plsc.subcore_barrier()

  pltpu.sync_copy(
      acc_sh.at[pl.ds(s * ROWS_PER_TILE, ROWS_PER_TILE)],
      out_hbm.at[c, pl.ds(s * ROWS_PER_TILE, ROWS_PER_TILE)])


def _unpack(p_ref, Hp):
  a = p_ref[0] + p_ref[1]
  return lax.slice(a, (0, 0), (N, Hp))


def _tc_mm_body(x_ref, w_ref, y_ref):
  y_ref[...] = jnp.dot(x_ref[...], w_ref[...],
                       preferred_element_type=jnp.float32)


def _mm(x, w, dout):
  return pl.pallas_call(
      _tc_mm_body,
      out_shape=jax.ShapeDtypeStruct((N, dout), jnp.float32),
  )(x, w)


def _tc_mid1_body(p_ref, pdeg_ref, r_ref, b1l_ref, b1r_ref, w2l_ref,
                  y2_ref, h_ref, inv_ref):
  agg = _unpack(p_ref, H)
  deg = lax.slice(_unpack(pdeg_ref, 16), (0, 0), (N, 1))
  inv = 1.0 / jnp.maximum(deg, 1.0)
  mean = agg * inv
  h = jnp.maximum(mean + b1l_ref[...] + r_ref[...] + b1r_ref[...], 0.0)
  y2_ref[...] = jnp.dot(h, w2l_ref[...], preferred_element_type=jnp.float32)
  h_ref[...] = h
  inv_ref[...] = inv


def _tc_mid2_body(p_ref, inv_ref, r_ref, b2l_ref, b2r_ref, w3lp_ref, w3r_ref,
                  y3p_ref, r3_ref):
  mean = _unpack(p_ref, H) * inv_ref[...]
  h = jnp.maximum(mean + b2l_ref[...] + r_ref[...] + b2r_ref[...], 0.0)
  y3p_ref[...] = jnp.dot(h, w3lp_ref[...], preferred_element_type=jnp.float32)
  r3_ref[...] = jnp.dot(h, w3r_ref[...], preferred_element_type=jnp.float32)


def _tc_final_body(p_ref, inv_ref, r3_ref, b3l_ref, b3r_ref, out_ref):
  val = lax.slice(_unpack(p_ref, 16), (0, 0), (N, 1))
  res = val * inv_ref[...] + b3l_ref[...] + r3_ref[...] + b3r_ref[...]
  out_ref[...] = jnp.reshape(res, (N,))


def kernel(x, edge_index, W1l, b1l, W1r, b1r, W2l, b2l, W2r, b2r,
           W3l, b3l, W3r, b3r):
  w3lp = jnp.pad(W3l, ((0, 0), (0, 15)))

  f32 = jnp.float32
  y1 = _mm(x, W1l, H)
  pdeg, ei_u = _deg(edge_index)

  p1 = _agg64_l1(ei_u, y1)
  r1 = _mm(x, W1r, H)

  y2, h1, inv = pl.pallas_call(
      _tc_mid1_body,
      out_shape=[jax.ShapeDtypeStruct((N, H), f32),
                 jax.ShapeDtypeStruct((N, H), f32),
                 jax.ShapeDtypeStruct((N, 1), f32)],
  )(p1, pdeg, r1, b1l, b1r, W2l)

  p2 = _agg64_l2(ei_u, y2)
  r2 = _mm(h1, W2r, H)

  y3p, r3 = pl.pallas_call(
      _tc_mid2_body,
      out_shape=[jax.ShapeDtypeStruct((N, 16), f32),
                 jax.ShapeDtypeStruct((N, 1), f32)],
  )(p2, inv, r2, b2l, b2r, w3lp, W3r)

  p3 = _agg16(ei_u, y3p)

  out = pl.pallas_call(
      _tc_final_body,
      out_shape=jax.ShapeDtypeStruct((N,), f32),
  )(p3, inv, r3, b3l, b3r)

  return out


# speedup vs baseline: 1.3069x; 1.0018x over previous
import functools

import jax
import jax.numpy as jnp
from jax import lax
from jax.experimental import pallas as pl
from jax.experimental.pallas import tpu as pltpu
from jax.experimental.pallas import tpu_sc as plsc

N = 10000
E = 320000
D_IN = 128
H = 64

NC = 2
NS = 16
NW = NC * NS
EPW = E // NW
NPAD = 10240
ROWS_PER_TILE = NPAD // NS

_SC_MESH = plsc.VectorSubcoreMesh(
    core_axis_name="c", subcore_axis_name="s", num_cores=NC, num_subcores=NS)


def _fill(buf, nrows, ncol16, vec):
  assert nrows % 4 == 0
  def frow(i, _):
    for u in range(4):
      for j in range(ncol16):
        buf[i * 4 + u, pl.ds(j * 16, 16)] = vec
    return 0
  lax.fori_loop(0, nrows // 4, frow, 0)


def _make_agg(Hp, CH):
  nch = EPW // CH
  NB = 3
  ngrp = (nch + NB - 1) // NB

  @functools.partial(
      pl.kernel,
      out_type=jax.ShapeDtypeStruct((NC, NPAD, Hp), jnp.float32),
      mesh=_SC_MESH,
      scratch_types=[
          pltpu.VMEM((NB, 2, CH), jnp.int32),
          pltpu.VMEM((CH, Hp), jnp.float32),
          pltpu.VMEM((CH, Hp), jnp.float32),
          pltpu.VMEM((CH, Hp), jnp.float32),
          pltpu.VMEM_SHARED((NPAD, Hp), jnp.float32),
          pltpu.SemaphoreType.DMA,
          pltpu.SemaphoreType.DMA,
          pltpu.SemaphoreType.DMA,
          pltpu.SemaphoreType.DMA,
          pltpu.SemaphoreType.DMA,
          pltpu.SemaphoreType.DMA,
      ],
      compiler_params=pltpu.CompilerParams(use_tc_tiling_on_sc=False),
  )
  def agg(ei_hbm, y_hbm, out_hbm, idx_v, rows0, rows1, rows2, acc_sh,
          gsem0, gsem1, gsem2, ssem0, ssem1, ssem2):
    rows = (rows0, rows1, rows2)
    gsems = (gsem0, gsem1, gsem2)
    ssems = (ssem0, ssem1, ssem2)
    c = lax.axis_index("c")
    s = lax.axis_index("s")
    wid = s * NC + c
    z16 = jnp.zeros((16,), jnp.float32)

    _fill(rows0, CH, Hp // 16, z16)
    _fill(rows1, ROWS_PER_TILE - CH, Hp // 16, z16)
    pltpu.sync_copy(rows0, acc_sh.at[pl.ds(s * ROWS_PER_TILE, CH)])
    pltpu.sync_copy(
        rows1.at[pl.ds(0, ROWS_PER_TILE - CH)],
        acc_sh.at[pl.ds(s * ROWS_PER_TILE + CH, ROWS_PER_TILE - CH)])
    plsc.subcore_barrier()

    base = wid * EPW

    def _stage(q, bq):
      off = base + q * CH
      pltpu.sync_copy(ei_hbm.at[:, pl.ds(off, CH)], idx_v.at[bq])
      pltpu.async_copy(y_hbm.at[idx_v.at[bq, 0]], rows[bq], gsems[bq],
                       priority=1)

    def _scatter_wait(b):
      pltpu.make_async_copy(rows[b], acc_sh.at[idx_v.at[b, 1]],
                            ssems[b]).wait()

    for q in range(NB - 1):
      _stage(q, q)

    def grp(k, _):
      for b in range(NB):
        g = NB * k + b

        @pl.when(g < nch)
        def _step():
          pltpu.make_async_copy(y_hbm.at[idx_v.at[b, 0]], rows[b],
                                gsems[b]).wait()
          pltpu.async_copy(rows[b], acc_sh.at[idx_v.at[b, 1]], ssems[b],
                           add=True)

          q = g + NB - 1
          bq = (b + NB - 1) % NB

          @pl.when(q < nch)
          def _prefetch():
            @pl.when(g >= 1)
            def _drain():
              _scatter_wait(bq)
            _stage(q, bq)

      return 0

    lax.fori_loop(0, ngrp, grp, 0)
    for b in range(NB):
      _scatter_wait(b)
    plsc.subcore_barrier()

    pltpu.sync_copy(
        acc_sh.at[pl.ds(s * ROWS_PER_TILE, ROWS_PER_TILE)],
        out_hbm.at[c, pl.ds(s * ROWS_PER_TILE, ROWS_PER_TILE)])

  return agg


_agg64_l1 = _make_agg(64, 400)
_agg64_l2 = _make_agg(64, 400)
_agg16 = _make_agg(16, 400)

_DEG_CH = 2000


@functools.partial(
    pl.kernel,
    out_type=[jax.ShapeDtypeStruct((NC, NPAD, 16), jnp.float32),
              jax.ShapeDtypeStruct((2, E), jnp.int32)],
    mesh=_SC_MESH,
    scratch_types=[
        pltpu.VMEM((2, _DEG_CH), jnp.int32),
        pltpu.VMEM((_DEG_CH, 16), jnp.float32),
        pltpu.VMEM_SHARED((NPAD, 16), jnp.float32),
    ],
    compiler_params=pltpu.CompilerParams(use_tc_tiling_on_sc=False),
)
def _deg(ei_hbm, out_hbm, eiu_hbm, idx2_v, ones_v, acc_sh):
  c = lax.axis_index("c")
  s = lax.axis_index("s")
  wid = s * NC + c

  z16 = jnp.zeros((16,), jnp.float32)
  _fill(ones_v, ROWS_PER_TILE, 1, z16)
  pltpu.sync_copy(
      ones_v.at[pl.ds(0, ROWS_PER_TILE)],
      acc_sh.at[pl.ds(s * ROWS_PER_TILE, ROWS_PER_TILE)])

  onehot = jnp.where(lax.iota(jnp.int32, 16) == 0, 1.0, 0.0)
  _fill(ones_v, _DEG_CH, 1, onehot)
  plsc.subcore_barrier()

  base = wid * EPW

  def chunk(t, _):
    off = base + t * _DEG_CH
    pltpu.sync_copy(ei_hbm.at[:, pl.ds(off, _DEG_CH)], idx2_v)
    pltpu.sync_copy(idx2_v, eiu_hbm.at[:, pl.ds(off, _DEG_CH)])
    pltpu.sync_copy(ones_v, acc_sh.at[idx2_v.at[1]], add=True)
    return 0

  lax.fori_loop(0, EPW // _DEG_CH, chunk, 0)
  plsc.subcore_barrier()

  pltpu.sync_copy(
      acc_sh.at[pl.ds(s * ROWS_PER_TILE, ROWS_PER_TILE)],
      out_hbm.at[c, pl.ds(s * ROWS_PER_TILE, ROWS_PER_TILE)])


def _unpack(p_ref, Hp):
  a = p_ref[0] + p_ref[1]
  return lax.slice(a, (0, 0), (N, Hp))


def _tc_mm_body(x_ref, w_ref, y_ref):
  y_ref[...] = jnp.dot(x_ref[...], w_ref[...],
                       preferred_element_type=jnp.float32)


def _mm(x, w, dout):
  return pl.pallas_call(
      _tc_mm_body,
      out_shape=jax.ShapeDtypeStruct((N, dout), jnp.float32),
  )(x, w)


def _tc_mid1_body(p_ref, pdeg_ref, r_ref, b1l_ref, b1r_ref, w2l_ref,
                  y2_ref, h_ref, inv_ref):
  agg = _unpack(p_ref, H)
  deg = lax.slice(_unpack(pdeg_ref, 16), (0, 0), (N, 1))
  inv = 1.0 / jnp.maximum(deg, 1.0)
  mean = agg * inv
  h = jnp.maximum(mean + b1l_ref[...] + r_ref[...] + b1r_ref[...], 0.0)
  y2_ref[...] = jnp.dot(h, w2l_ref[...], preferred_element_type=jnp.float32)
  h_ref[...] = h
  inv_ref[...] = inv


def _tc_mid2_body(p_ref, inv_ref, r_ref, b2l_ref, b2r_ref, w3lp_ref, w3r_ref,
                  y3p_ref, r3_ref):
  mean = _unpack(p_ref, H) * inv_ref[...]
  h = jnp.maximum(mean + b2l_ref[...] + r_ref[...] + b2r_ref[...], 0.0)
  y3p_ref[...] = jnp.dot(h, w3lp_ref[...], preferred_element_type=jnp.float32)
  r3_ref[...] = jnp.dot(h, w3r_ref[...], preferred_element_type=jnp.float32)


def _tc_final_body(p_ref, inv_ref, r3_ref, b3l_ref, b3r_ref, out_ref):
  val = lax.slice(_unpack(p_ref, 16), (0, 0), (N, 1))
  res = val * inv_ref[...] + b3l_ref[...] + r3_ref[...] + b3r_ref[...]
  out_ref[...] = jnp.reshape(res, (N,))


def kernel(x, edge_index, W1l, b1l, W1r, b1r, W2l, b2l, W2r, b2r,
           W3l, b3l, W3r, b3r):
  w3lp = jnp.pad(W3l, ((0, 0), (0, 15)))

  f32 = jnp.float32
  y1 = _mm(x, W1l, H)
  pdeg, ei_u = _deg(edge_index)

  p1 = _agg64_l1(ei_u, y1)
  r1 = _mm(x, W1r, H)

  y2, h1, inv = pl.pallas_call(
      _tc_mid1_body,
      out_shape=[jax.ShapeDtypeStruct((N, H), f32),
                 jax.ShapeDtypeStruct((N, H), f32),
                 jax.ShapeDtypeStruct((N, 1), f32)],
  )(p1, pdeg, r1, b1l, b1r, W2l)

  p2 = _agg64_l2(ei_u, y2)
  r2 = _mm(h1, W2r, H)

  y3p, r3 = pl.pallas_call(
      _tc_mid2_body,
      out_shape=[jax.ShapeDtypeStruct((N, 16), f32),
                 jax.ShapeDtypeStruct((N, 1), f32)],
  )(p2, inv, r2, b2l, b2r, w3lp, W3r)

  p3 = _agg16(ei_u, y3p)

  out = pl.pallas_call(
      _tc_final_body,
      out_shape=jax.ShapeDtypeStruct((N,), f32),
  )(p3, inv, r3, b3l, b3r)

  return out
